# baseline (speedup 1.0000x reference)
import functools

import jax
import jax.numpy as jnp
from jax import lax
from jax.experimental import pallas as pl
from jax.experimental.pallas import tpu as pltpu
from jax.experimental.pallas import tpu_sc as plsc

LENGTH = 2048
N_PAIRS = 16
FLAT = 2 * N_PAIRS

_mesh = plsc.ScalarSubcoreMesh(axis_name="c", num_cores=1)


@functools.partial(
    pl.kernel,
    mesh=_mesh,
    out_type=jax.ShapeDtypeStruct((N_PAIRS, 2), jnp.int32),
    scratch_types=[pltpu.SMEM((N_PAIRS, 2), jnp.int32)],
)
def _trunc_sc(idx_hbm, out_hbm, scratch):
    cid = lax.axis_index("c")

    @pl.when(cid == 0)
    def _():
        pltpu.sync_copy(idx_hbm, scratch)
        for i in range(N_PAIRS):
            b = scratch[i, 0]
            e = scratch[i, 1]
            scratch[i, 1] = jnp.minimum(b + LENGTH, e)
        pltpu.sync_copy(scratch, out_hbm)


_COPY_BLOCK = 2048


def _copy_body(x_ref, o_ref):
    o_ref[...] = x_ref[...]


def _tc_copy(values):
    rows, cols = values.shape
    return pl.pallas_call(
        _copy_body,
        grid=(rows // _COPY_BLOCK,),
        in_specs=[pl.BlockSpec((_COPY_BLOCK, cols), lambda i: (i, 0))],
        out_specs=pl.BlockSpec((_COPY_BLOCK, cols), lambda i: (i, 0)),
        out_shape=jax.ShapeDtypeStruct(values.shape, values.dtype),
    )(values)


def kernel(values, indices):
    vals_out = _tc_copy(values)
    out = _trunc_sc(indices)
    return (vals_out, out)


# speedup vs baseline: 1.0077x; 1.0077x over previous
import functools

import jax
import jax.numpy as jnp
from jax import lax
from jax.experimental import pallas as pl
from jax.experimental.pallas import tpu as pltpu
from jax.experimental.pallas import tpu_sc as plsc

LENGTH = 2048
N_PAIRS = 16
FLAT = 2 * N_PAIRS

_mesh = plsc.ScalarSubcoreMesh(axis_name="c", num_cores=1)


@functools.partial(
    pl.kernel,
    mesh=_mesh,
    out_type=jax.ShapeDtypeStruct((N_PAIRS, 2), jnp.int32),
    scratch_types=[pltpu.SMEM((N_PAIRS, 2), jnp.int32)],
)
def _trunc_sc(idx_hbm, out_hbm, scratch):
    cid = lax.axis_index("c")

    @pl.when(cid == 0)
    def _():
        pltpu.sync_copy(idx_hbm, scratch)
        for i in range(N_PAIRS):
            b = scratch[i, 0]
            e = scratch[i, 1]
            scratch[i, 1] = jnp.minimum(b + LENGTH, e)
        pltpu.sync_copy(scratch, out_hbm)


_COPY_BLOCK = 2048
_HEAD_BLOCKS = 3


def _copy_body(x_ref, o_ref):
    o_ref[...] = x_ref[...]


def _copy_tail_body(_, x_ref, o_ref):
    o_ref[...] = x_ref[...]


def _tc_copy_head(values):
    rows, cols = values.shape
    return pl.pallas_call(
        _copy_body,
        grid=(_HEAD_BLOCKS,),
        in_specs=[pl.BlockSpec((_COPY_BLOCK, cols), lambda i: (i, 0))],
        out_specs=pl.BlockSpec((_COPY_BLOCK, cols), lambda i: (i, 0)),
        out_shape=jax.ShapeDtypeStruct(values.shape, values.dtype),
    )(values)


def _tc_copy_tail(partial, values):
    rows, cols = values.shape
    nblocks = rows // _COPY_BLOCK - _HEAD_BLOCKS
    return pl.pallas_call(
        _copy_tail_body,
        grid=(nblocks,),
        in_specs=[
            pl.BlockSpec(memory_space=pl.ANY),
            pl.BlockSpec((_COPY_BLOCK, cols), lambda i: (i + _HEAD_BLOCKS, 0)),
        ],
        out_specs=pl.BlockSpec((_COPY_BLOCK, cols), lambda i: (i + _HEAD_BLOCKS, 0)),
        out_shape=jax.ShapeDtypeStruct(values.shape, values.dtype),
        input_output_aliases={0: 0},
    )(partial, values)


def kernel(values, indices):
    partial = _tc_copy_head(values)
    partial, indices = lax.optimization_barrier((partial, indices))
    out = _trunc_sc(indices)
    vals_out = _tc_copy_tail(partial, values)
    return (vals_out, out)
